# baseline (speedup 1.0000x reference)
import functools

import jax
import jax.numpy as jnp
from jax import lax
from jax.experimental import pallas as pl
from jax.experimental.pallas import tpu as pltpu
from jax.experimental.pallas import tpu_sc as plsc

NC = 2
NS = 16
NW = NC * NS
LANES = 16
NBUF = 4


def _sc_relayout(table_t, tail_pad):
    e_dim, v_dim = table_t.shape
    vt_sz = 128
    n_full = v_dim // vt_sz
    rem = v_dim - n_full * vt_sz
    per_w = (n_full + NW - 1) // NW
    vecs_e = e_dim // LANES

    mesh = plsc.VectorSubcoreMesh(core_axis_name="c", subcore_axis_name="s")

    @functools.partial(
        pl.kernel,
        mesh=mesh,
        compiler_params=pltpu.CompilerParams(
            use_tc_tiling_on_sc=True, needs_layout_passes=False),
        out_type=jax.ShapeDtypeStruct((v_dim, 2 * e_dim), jnp.float32),
        scratch_types=[
            [pltpu.VMEM((e_dim, vt_sz), jnp.float32) for _ in range(2)],
            [pltpu.VMEM((vt_sz, 2 * e_dim), jnp.float32) for _ in range(2)],
            [pltpu.SemaphoreType.DMA for _ in range(2)],
            [pltpu.SemaphoreType.DMA for _ in range(2)],
        ],
    )
    def a(tt_hbm, tail_hbm, out_hbm, srcs, dsts, isems, osems):
        wid = lax.axis_index("s") * NC + lax.axis_index("c")
        base = wid * per_w
        iota = lax.iota(jnp.int32, 16)

        def tile_ok(j):
            if isinstance(j, int) and j >= per_w:
                return jnp.bool_(False)
            return jnp.logical_and(j < per_w, base + j < n_full)

        def in_issue(j, b):
            v0 = pl.multiple_of((base + j) * vt_sz, 128)
            pltpu.async_copy(
                tt_hbm.at[pl.ds(0, e_dim), pl.ds(v0, vt_sz)], srcs[b],
                isems[b])

        def in_wait(j, b):
            v0 = pl.multiple_of((base + j) * vt_sz, 128)
            pltpu.make_async_copy(
                tt_hbm.at[pl.ds(0, e_dim), pl.ds(v0, vt_sz)], srcs[b],
                isems[b]).wait()

        def out_issue(j, b):
            v0 = pl.multiple_of((base + j) * vt_sz, 128)
            pltpu.async_copy(dsts[b], out_hbm.at[pl.ds(v0, vt_sz)], osems[b])

        def out_wait(j, b):
            v0 = pl.multiple_of((base + j) * vt_sz, 128)
            pltpu.make_async_copy(
                dsts[b], out_hbm.at[pl.ds(v0, vt_sz)], osems[b]).wait()

        def transpose(b):
            src, dst = srcs[b], dsts[b]

            @plsc.parallel_loop(0, vt_sz, 1, unroll=4)
            def _(l):
                v_idx = jnp.full((16,), l, dtype=jnp.int32)
                for k in range(vecs_e):
                    vals = plsc.load_gather(src, [iota + (16 * k), v_idx])
                    dst[l, pl.ds(16 * k, 16)] = vals

        @pl.when(tile_ok(0))
        def _():
            in_issue(0, 0)

        def macro(m, carry):
            for i in range(2):
                j = 2 * m + i

                @pl.when(tile_ok(j))
                def _():
                    in_wait(j, i)

                    @pl.when(tile_ok(j + 1))
                    def _():
                        in_issue(j + 1, 1 - i)

                    @pl.when(j >= 2)
                    def _():
                        out_wait(j - 2, i)

                    transpose(i)
                    out_issue(j, i)
            return carry

        lax.fori_loop(0, (per_w + 1) // 2, macro, 0)

        jm = jnp.minimum(per_w - 1, n_full - 1 - base)
        for i in range(2):
            j_i = jm - ((jm - i) % 2)

            @pl.when(j_i >= 0)
            def _():
                out_wait(j_i, i)

        @pl.when(wid == 0)
        def _():
            pltpu.sync_copy(tail_hbm, srcs[0])
            pltpu.sync_copy(srcs[0], out_hbm.at[pl.ds(n_full * vt_sz, rem)])

    return a(table_t, tail_pad)


def _sc_embed(x2d, tok_pad, pos_table):
    bsz, t_len = x2d.shape
    v_dim, row_w = tok_pad.shape
    _, emb = pos_table.shape

    rows_w = bsz // NW
    chunk = t_len
    n_chunk = rows_w
    vecs_per_row = emb // LANES

    sub_sizes = []
    off = 0
    while off < chunk:
        n = min(128, chunk - off)
        sub_sizes.append((off, n))
        off += n

    mesh = plsc.VectorSubcoreMesh(core_axis_name="c", subcore_axis_name="s")

    @functools.partial(
        pl.kernel,
        mesh=mesh,
        compiler_params=pltpu.CompilerParams(use_tc_tiling_on_sc=False),
        out_type=jax.ShapeDtypeStruct((bsz, t_len, row_w), jnp.float32),
        scratch_types=[
            [pltpu.VMEM((chunk,), jnp.int32) for _ in range(NBUF)],
            pltpu.VMEM((t_len, emb), jnp.float32),
            [pltpu.VMEM((chunk, row_w), jnp.float32) for _ in range(NBUF)],
            [pltpu.SemaphoreType.DMA for _ in range(NBUF)],
            [pltpu.SemaphoreType.DMA for _ in range(NBUF)],
            [pltpu.SemaphoreType.DMA for _ in range(NBUF)],
        ],
    )
    def k(idx_hbm, tok_hbm, pos_hbm, out_hbm, ibufs, pos_v, bufs,
          isems, gsems, ssems):
        wid = lax.axis_index("s") * NC + lax.axis_index("c")
        base_row = wid * rows_w
        pltpu.sync_copy(pos_hbm, pos_v)

        def i_issue(r, ib, sem):
            pltpu.async_copy(idx_hbm.at[base_row + r], ib, sem)

        def i_wait(r, ib, sem):
            pltpu.make_async_copy(idx_hbm.at[base_row + r], ib, sem).wait()

        def g_issue(ib, buf, sem):
            for so, sn in sub_sizes:
                pltpu.async_copy(
                    tok_hbm.at[ib.at[pl.ds(so, sn)]],
                    buf.at[pl.ds(so, sn)], sem)

        def g_wait(ib, buf, sem):
            for so, sn in sub_sizes:
                pltpu.make_async_copy(
                    tok_hbm.at[ib.at[pl.ds(so, sn)]],
                    buf.at[pl.ds(so, sn)], sem).wait()

        def s_issue(r, buf, sem):
            pltpu.async_copy(buf, out_hbm.at[base_row + r], sem)

        def s_wait(r, buf, sem):
            pltpu.make_async_copy(buf, out_hbm.at[base_row + r], sem).wait()

        def add_pos(buf):
            @plsc.parallel_loop(0, chunk, 1, unroll=8)
            def _(j):
                for v in range(vecs_per_row):
                    sl = pl.ds(v * LANES, LANES)
                    plsc.addupdate(buf.at[j, sl], pos_v[j, sl])

        for i in range(NBUF):
            i_issue(i, ibufs[i], isems[i])
        for i in range(2):
            i_wait(i, ibufs[i], isems[i])
            g_issue(ibufs[i], bufs[i], gsems[i])

        def macro(m, carry):
            for i in range(NBUF):
                g = NBUF * m + i
                g_wait(ibufs[i], bufs[i], gsems[i])

                @pl.when(g + NBUF < n_chunk)
                def _():
                    i_issue(g + NBUF, ibufs[i], isems[i])

                add_pos(bufs[i])
                s_issue(g, bufs[i], ssems[i])
                i2 = (i + 2) % NBUF
                g2 = g + 2

                @pl.when(g2 < n_chunk)
                def _():
                    @pl.when(g2 >= NBUF)
                    def _():
                        s_wait(g - 2, bufs[i2], ssems[i2])
                    i_wait(g2, ibufs[i2], isems[i2])
                    g_issue(ibufs[i2], bufs[i2], gsems[i2])
            return carry

        lax.fori_loop(0, n_chunk // NBUF, macro, 0)

        for i in range(NBUF):
            s_wait(n_chunk - NBUF + i, bufs[i], ssems[i])

    return k(x2d, tok_pad, pos_table)


def kernel(x, token_table, pos_table):
    v_dim, emb = token_table.shape
    rem = v_dim % 128
    tail_pad = jnp.pad(token_table[v_dim - rem:], ((0, 0), (0, 2 * emb - emb)))
    tok_pad = _sc_relayout(token_table.T, tail_pad)
    out = _sc_embed(x.astype(jnp.int32), tok_pad, pos_table)
    return out[:, :, :emb]


# speedup vs baseline: 1.6214x; 1.6214x over previous
import functools

import jax
import jax.numpy as jnp
from jax import lax
from jax.experimental import pallas as pl
from jax.experimental.pallas import tpu as pltpu
from jax.experimental.pallas import tpu_sc as plsc

NC = 2
NS = 16
NW = NC * NS
LANES = 16
NBUF = 4


def _sc_relayout(table_t, tail_pad):
    e_dim, v_dim = table_t.shape
    vt_sz = 128
    n_full = v_dim // vt_sz
    rem = v_dim - n_full * vt_sz
    per_w = (n_full + NW - 1) // NW
    vecs_e = e_dim // LANES

    mesh = plsc.VectorSubcoreMesh(core_axis_name="c", subcore_axis_name="s")

    @functools.partial(
        pl.kernel,
        mesh=mesh,
        compiler_params=pltpu.CompilerParams(
            use_tc_tiling_on_sc=True, needs_layout_passes=False),
        out_type=jax.ShapeDtypeStruct((v_dim, 2 * e_dim), jnp.float32),
        scratch_types=[
            [pltpu.VMEM((e_dim, vt_sz), jnp.float32) for _ in range(2)],
            [pltpu.VMEM((vt_sz, 2 * e_dim), jnp.float32) for _ in range(2)],
            [pltpu.SemaphoreType.DMA for _ in range(2)],
            [pltpu.SemaphoreType.DMA for _ in range(2)],
        ],
    )
    def a(tt_hbm, tail_hbm, out_hbm, srcs, dsts, isems, osems):
        wid = lax.axis_index("s") * NC + lax.axis_index("c")
        base = wid * per_w
        iota = lax.iota(jnp.int32, 16)

        def tile_ok(j):
            if isinstance(j, int) and j >= per_w:
                return jnp.bool_(False)
            return jnp.logical_and(j < per_w, base + j < n_full)

        def in_issue(j, b):
            v0 = pl.multiple_of((base + j) * vt_sz, 128)
            pltpu.async_copy(
                tt_hbm.at[pl.ds(0, e_dim), pl.ds(v0, vt_sz)], srcs[b],
                isems[b])

        def in_wait(j, b):
            v0 = pl.multiple_of((base + j) * vt_sz, 128)
            pltpu.make_async_copy(
                tt_hbm.at[pl.ds(0, e_dim), pl.ds(v0, vt_sz)], srcs[b],
                isems[b]).wait()

        def out_issue(j, b):
            v0 = pl.multiple_of((base + j) * vt_sz, 128)
            pltpu.async_copy(dsts[b], out_hbm.at[pl.ds(v0, vt_sz)], osems[b])

        def out_wait(j, b):
            v0 = pl.multiple_of((base + j) * vt_sz, 128)
            pltpu.make_async_copy(
                dsts[b], out_hbm.at[pl.ds(v0, vt_sz)], osems[b]).wait()

        def transpose(b):
            src, dst = srcs[b], dsts[b]

            @plsc.parallel_loop(0, vt_sz, 1, unroll=4)
            def _(l):
                v_idx = jnp.bitwise_and(iota + l, vt_sz - 1)
                for k in range(vecs_e):
                    e_idx = iota + (16 * k)
                    vals = plsc.load_gather(src, [e_idx, v_idx])
                    plsc.store_scatter(dst, [v_idx, e_idx], vals)

        @pl.when(tile_ok(0))
        def _():
            in_issue(0, 0)

        def macro(m, carry):
            for i in range(2):
                j = 2 * m + i

                @pl.when(tile_ok(j))
                def _():
                    in_wait(j, i)

                    @pl.when(tile_ok(j + 1))
                    def _():
                        in_issue(j + 1, 1 - i)

                    @pl.when(j >= 2)
                    def _():
                        out_wait(j - 2, i)

                    transpose(i)
                    out_issue(j, i)
            return carry

        lax.fori_loop(0, (per_w + 1) // 2, macro, 0)

        jm = jnp.minimum(per_w - 1, n_full - 1 - base)
        for i in range(2):
            j_i = jm - ((jm - i) % 2)

            @pl.when(j_i >= 0)
            def _():
                out_wait(j_i, i)

        @pl.when(wid == 0)
        def _():
            pltpu.sync_copy(tail_hbm, srcs[0])
            pltpu.sync_copy(srcs[0], out_hbm.at[pl.ds(n_full * vt_sz, rem)])

    return a(table_t, tail_pad)


def _sc_embed(x2d, tok_pad, pos_table):
    bsz, t_len = x2d.shape
    v_dim, row_w = tok_pad.shape
    _, emb = pos_table.shape

    rows_w = bsz // NW
    chunk = t_len
    n_chunk = rows_w
    vecs_per_row = emb // LANES

    sub_sizes = []
    off = 0
    while off < chunk:
        n = min(128, chunk - off)
        sub_sizes.append((off, n))
        off += n

    mesh = plsc.VectorSubcoreMesh(core_axis_name="c", subcore_axis_name="s")

    @functools.partial(
        pl.kernel,
        mesh=mesh,
        compiler_params=pltpu.CompilerParams(use_tc_tiling_on_sc=False),
        out_type=jax.ShapeDtypeStruct((bsz, t_len, row_w), jnp.float32),
        scratch_types=[
            [pltpu.VMEM((chunk,), jnp.int32) for _ in range(NBUF)],
            pltpu.VMEM((t_len, emb), jnp.float32),
            [pltpu.VMEM((chunk, row_w), jnp.float32) for _ in range(NBUF)],
            [pltpu.SemaphoreType.DMA for _ in range(NBUF)],
            [pltpu.SemaphoreType.DMA for _ in range(NBUF)],
            [pltpu.SemaphoreType.DMA for _ in range(NBUF)],
        ],
    )
    def k(idx_hbm, tok_hbm, pos_hbm, out_hbm, ibufs, pos_v, bufs,
          isems, gsems, ssems):
        wid = lax.axis_index("s") * NC + lax.axis_index("c")
        base_row = wid * rows_w
        pltpu.sync_copy(pos_hbm, pos_v)

        def i_issue(r, ib, sem):
            pltpu.async_copy(idx_hbm.at[base_row + r], ib, sem)

        def i_wait(r, ib, sem):
            pltpu.make_async_copy(idx_hbm.at[base_row + r], ib, sem).wait()

        def g_issue(ib, buf, sem):
            for so, sn in sub_sizes:
                pltpu.async_copy(
                    tok_hbm.at[ib.at[pl.ds(so, sn)]],
                    buf.at[pl.ds(so, sn)], sem)

        def g_wait(ib, buf, sem):
            for so, sn in sub_sizes:
                pltpu.make_async_copy(
                    tok_hbm.at[ib.at[pl.ds(so, sn)]],
                    buf.at[pl.ds(so, sn)], sem).wait()

        def s_issue(r, buf, sem):
            pltpu.async_copy(
                buf.at[pl.ds(0, chunk), pl.ds(0, emb)],
                out_hbm.at[base_row + r, pl.ds(0, t_len), pl.ds(0, emb)], sem)

        def s_wait(r, buf, sem):
            pltpu.make_async_copy(
                buf.at[pl.ds(0, chunk), pl.ds(0, emb)],
                out_hbm.at[base_row + r, pl.ds(0, t_len), pl.ds(0, emb)],
                sem).wait()

        def add_pos(buf):
            @plsc.parallel_loop(0, chunk, 1, unroll=8)
            def _(j):
                for v in range(vecs_per_row):
                    sl = pl.ds(v * LANES, LANES)
                    plsc.addupdate(buf.at[j, sl], pos_v[j, sl])

        for i in range(NBUF):
            i_issue(i, ibufs[i], isems[i])
        for i in range(2):
            i_wait(i, ibufs[i], isems[i])
            g_issue(ibufs[i], bufs[i], gsems[i])

        def macro(m, carry):
            for i in range(NBUF):
                g = NBUF * m + i
                g_wait(ibufs[i], bufs[i], gsems[i])

                @pl.when(g + NBUF < n_chunk)
                def _():
                    i_issue(g + NBUF, ibufs[i], isems[i])

                add_pos(bufs[i])
                s_issue(g, bufs[i], ssems[i])
                i2 = (i + 2) % NBUF
                g2 = g + 2

                @pl.when(g2 < n_chunk)
                def _():
                    @pl.when(g2 >= NBUF)
                    def _():
                        s_wait(g - 2, bufs[i2], ssems[i2])
                    i_wait(g2, ibufs[i2], isems[i2])
                    g_issue(ibufs[i2], bufs[i2], gsems[i2])
            return carry

        lax.fori_loop(0, n_chunk // NBUF, macro, 0)

        for i in range(NBUF):
            s_wait(n_chunk - NBUF + i, bufs[i], ssems[i])

    return k(x2d, tok_pad, pos_table)


def kernel(x, token_table, pos_table):
    v_dim, emb = token_table.shape
    rem = v_dim % 128
    tail_pad = jnp.pad(token_table[v_dim - rem:], ((0, 0), (0, 2 * emb - emb)))
    tok_pad = _sc_relayout(token_table.T, tail_pad)
    out = _sc_embed(x.astype(jnp.int32), tok_pad, pos_table)
    return out[:, :, :emb]


# speedup vs baseline: 1.9519x; 1.2039x over previous
import functools

import jax
import jax.numpy as jnp
from jax import lax
from jax.experimental import pallas as pl
from jax.experimental.pallas import tpu as pltpu
from jax.experimental.pallas import tpu_sc as plsc

NC = 2
NS = 16
NW = NC * NS
LANES = 16
NBUF = 4


def _sc_relayout(table_t, tail_pad):
    e_dim, v_dim = table_t.shape
    vt_sz = 128
    n_full = v_dim // vt_sz
    rem = v_dim - n_full * vt_sz
    per_w = (n_full + NW - 1) // NW
    vecs_e = e_dim // LANES

    mesh = plsc.VectorSubcoreMesh(core_axis_name="c", subcore_axis_name="s")

    @functools.partial(
        pl.kernel,
        mesh=mesh,
        compiler_params=pltpu.CompilerParams(
            use_tc_tiling_on_sc=True, needs_layout_passes=False),
        out_type=jax.ShapeDtypeStruct((v_dim // 2, vt_sz), jnp.float32),
        scratch_types=[
            [pltpu.VMEM((e_dim, vt_sz), jnp.float32) for _ in range(2)],
            [pltpu.VMEM((vt_sz // 2, vt_sz), jnp.float32) for _ in range(2)],
            [pltpu.SemaphoreType.DMA for _ in range(2)],
            [pltpu.SemaphoreType.DMA for _ in range(2)],
        ],
    )
    def a(tt_hbm, tail_hbm, out_hbm, srcs, dsts, isems, osems):
        wid = lax.axis_index("s") * NC + lax.axis_index("c")
        base = wid * per_w
        iota = lax.iota(jnp.int32, 16)

        def tile_ok(j):
            if isinstance(j, int) and j >= per_w:
                return jnp.bool_(False)
            return jnp.logical_and(j < per_w, base + j < n_full)

        def in_issue(j, b):
            v0 = pl.multiple_of((base + j) * vt_sz, 128)
            pltpu.async_copy(
                tt_hbm.at[pl.ds(0, e_dim), pl.ds(v0, vt_sz)], srcs[b],
                isems[b])

        def in_wait(j, b):
            v0 = pl.multiple_of((base + j) * vt_sz, 128)
            pltpu.make_async_copy(
                tt_hbm.at[pl.ds(0, e_dim), pl.ds(v0, vt_sz)], srcs[b],
                isems[b]).wait()

        def out_issue(j, b):
            r0 = pl.multiple_of((base + j) * (vt_sz // 2), 8)
            pltpu.async_copy(dsts[b], out_hbm.at[pl.ds(r0, vt_sz // 2)],
                             osems[b])

        def out_wait(j, b):
            r0 = pl.multiple_of((base + j) * (vt_sz // 2), 8)
            pltpu.make_async_copy(
                dsts[b], out_hbm.at[pl.ds(r0, vt_sz // 2)], osems[b]).wait()

        def transpose(b):
            src, dst = srcs[b], dsts[b]

            @plsc.parallel_loop(0, vt_sz, 1, unroll=4)
            def _(l):
                v_idx = jnp.bitwise_and(iota + l, vt_sz - 1)
                for k in range(vecs_e):
                    e_idx = iota + (16 * k)
                    vals = plsc.load_gather(src, [e_idx, v_idx])
                    plsc.store_scatter(
                        dst,
                        [v_idx >> 1,
                         jnp.bitwise_and(v_idx, 1) * e_dim + e_idx],
                        vals)

        @pl.when(tile_ok(0))
        def _():
            in_issue(0, 0)

        def macro(m, carry):
            for i in range(2):
                j = 2 * m + i

                @pl.when(tile_ok(j))
                def _():
                    in_wait(j, i)

                    @pl.when(tile_ok(j + 1))
                    def _():
                        in_issue(j + 1, 1 - i)

                    @pl.when(j >= 2)
                    def _():
                        out_wait(j - 2, i)

                    transpose(i)
                    out_issue(j, i)
            return carry

        lax.fori_loop(0, (per_w + 1) // 2, macro, 0)

        jm = jnp.minimum(per_w - 1, n_full - 1 - base)
        for i in range(2):
            j_i = jm - ((jm - i) % 2)

            @pl.when(j_i >= 0)
            def _():
                out_wait(j_i, i)

        @pl.when(wid == 0)
        def _():
            pltpu.sync_copy(tail_hbm, srcs[0].at[pl.ds(0, rem // 2)])
            pltpu.sync_copy(
                srcs[0].at[pl.ds(0, rem // 2)],
                out_hbm.at[pl.ds(n_full * (vt_sz // 2), rem // 2)])

    return a(table_t, tail_pad)


def _sc_embed(x2d, tok_rm, pos_table):
    bsz, t_len = x2d.shape
    v_dim, row_w = tok_rm.shape
    _, emb = pos_table.shape

    rows_w = bsz // NW
    chunk = t_len
    n_chunk = rows_w
    vecs_per_row = emb // LANES

    sub_sizes = []
    off = 0
    while off < chunk:
        n = min(128, chunk - off)
        sub_sizes.append((off, n))
        off += n

    mesh = plsc.VectorSubcoreMesh(core_axis_name="c", subcore_axis_name="s")

    @functools.partial(
        pl.kernel,
        mesh=mesh,
        compiler_params=pltpu.CompilerParams(use_tc_tiling_on_sc=False),
        out_type=jax.ShapeDtypeStruct((bsz, t_len, 2 * emb), jnp.float32),
        scratch_types=[
            [pltpu.VMEM((chunk,), jnp.int32) for _ in range(NBUF)],
            pltpu.VMEM((t_len, emb), jnp.float32),
            [pltpu.VMEM((chunk, emb), jnp.float32) for _ in range(NBUF)],
            [pltpu.SemaphoreType.DMA for _ in range(NBUF)],
            [pltpu.SemaphoreType.DMA for _ in range(NBUF)],
            [pltpu.SemaphoreType.DMA for _ in range(NBUF)],
        ],
    )
    def k(idx_hbm, tok_hbm, pos_hbm, out_hbm, ibufs, pos_v, bufs,
          isems, gsems, ssems):
        wid = lax.axis_index("s") * NC + lax.axis_index("c")
        base_row = wid * rows_w
        pltpu.sync_copy(pos_hbm, pos_v)

        def i_issue(r, ib, sem):
            pltpu.async_copy(idx_hbm.at[base_row + r], ib, sem)

        def i_wait(r, ib, sem):
            pltpu.make_async_copy(idx_hbm.at[base_row + r], ib, sem).wait()

        def g_issue(ib, buf, sem):
            for so, sn in sub_sizes:
                pltpu.async_copy(
                    tok_hbm.at[ib.at[pl.ds(so, sn)]],
                    buf.at[pl.ds(so, sn)], sem)

        def g_wait(ib, buf, sem):
            for so, sn in sub_sizes:
                pltpu.make_async_copy(
                    tok_hbm.at[ib.at[pl.ds(so, sn)]],
                    buf.at[pl.ds(so, sn)], sem).wait()

        def s_issue(r, buf, sem):
            pltpu.async_copy(
                buf.at[pl.ds(0, chunk), pl.ds(0, emb)],
                out_hbm.at[base_row + r, pl.ds(0, t_len), pl.ds(0, emb)], sem)

        def s_wait(r, buf, sem):
            pltpu.make_async_copy(
                buf.at[pl.ds(0, chunk), pl.ds(0, emb)],
                out_hbm.at[base_row + r, pl.ds(0, t_len), pl.ds(0, emb)],
                sem).wait()

        def add_pos(buf):
            @plsc.parallel_loop(0, chunk, 1, unroll=8)
            def _(j):
                for v in range(vecs_per_row):
                    sl = pl.ds(v * LANES, LANES)
                    plsc.addupdate(buf.at[j, sl], pos_v[j, sl])

        for i in range(NBUF):
            i_issue(i, ibufs[i], isems[i])
        for i in range(2):
            i_wait(i, ibufs[i], isems[i])
            g_issue(ibufs[i], bufs[i], gsems[i])

        def macro(m, carry):
            for i in range(NBUF):
                g = NBUF * m + i
                g_wait(ibufs[i], bufs[i], gsems[i])

                @pl.when(g + NBUF < n_chunk)
                def _():
                    i_issue(g + NBUF, ibufs[i], isems[i])

                add_pos(bufs[i])
                s_issue(g, bufs[i], ssems[i])
                i2 = (i + 2) % NBUF
                g2 = g + 2

                @pl.when(g2 < n_chunk)
                def _():
                    @pl.when(g2 >= NBUF)
                    def _():
                        s_wait(g - 2, bufs[i2], ssems[i2])
                    i_wait(g2, ibufs[i2], isems[i2])
                    g_issue(ibufs[i2], bufs[i2], gsems[i2])
            return carry

        lax.fori_loop(0, n_chunk // NBUF, macro, 0)

        for i in range(NBUF):
            s_wait(n_chunk - NBUF + i, bufs[i], ssems[i])

    return k(x2d, tok_rm, pos_table)


def kernel(x, token_table, pos_table):
    v_dim, emb = token_table.shape
    rem = v_dim % 128
    tail_c = token_table[v_dim - rem:].reshape(rem // 2, 2 * emb)
    tok_c = _sc_relayout(token_table.T, tail_c)
    tok_rm = tok_c.reshape(v_dim, emb)
    out = _sc_embed(x.astype(jnp.int32), tok_rm, pos_table)
    return out[:, :, :emb]
